# baseline (speedup 1.0000x reference)
import jax
import jax.numpy as jnp
from jax import lax
from jax.experimental import pallas as pl

_EPS = 1e-06
_LT = 1024


def _split_bf16(x):
    hi = x.astype(jnp.bfloat16)
    lo = (x - hi.astype(jnp.float32)).astype(jnp.bfloat16)
    return hi, lo


def _body(xt_ref, out_ref, qt0_ref, qt0t_ref, ratet_ref, o_ref):
    lt, s = out_ref.shape[1], out_ref.shape[2]
    f32 = jnp.float32
    xt = xt_ref[0, 0, :]
    iota = lax.broadcasted_iota(jnp.int32, (lt, s), 1)
    oh16 = (xt[:, None] == iota).astype(jnp.bfloat16)

    recip = 1.0 / (qt0t_ref[0] + _EPS)
    diag = lax.broadcasted_iota(jnp.int32, (s, s), 0) == \
        lax.broadcasted_iota(jnp.int32, (s, s), 1)
    ratez = jnp.where(diag, 0.0, ratet_ref[0])
    r_hi, r_lo = _split_bf16(recip)
    z_hi, z_lo = _split_bf16(ratez)

    def sel(tab_hi, tab_lo):
        a = jnp.dot(oh16, tab_hi, preferred_element_type=f32)
        b = jnp.dot(oh16, tab_lo, preferred_element_type=f32)
        return a + b

    recip_g = sel(r_hi, r_lo)
    fwd_g = sel(z_hi, z_lo)
    d16 = (out_ref[0] * recip_g).astype(jnp.bfloat16)
    score = jnp.dot(d16, qt0_ref[0].astype(jnp.bfloat16),
                    preferred_element_type=f32)
    o_ref[0] = fwd_g * score


def kernel(output, xt, t, qt0, rate):
    del t
    b, l, s = output.shape
    nb = l // _LT
    xt3 = xt.reshape(b * nb, 1, _LT)
    qt0t = qt0.swapaxes(1, 2)
    ratet = rate.swapaxes(1, 2)
    return pl.pallas_call(
        _body,
        grid=(b, nb),
        in_specs=[
            pl.BlockSpec((1, 1, _LT), lambda bi, li: (bi * nb + li, 0, 0)),
            pl.BlockSpec((1, _LT, s), lambda bi, li: (bi, li, 0)),
            pl.BlockSpec((1, s, s), lambda bi, li: (bi, 0, 0)),
            pl.BlockSpec((1, s, s), lambda bi, li: (bi, 0, 0)),
            pl.BlockSpec((1, s, s), lambda bi, li: (bi, 0, 0)),
        ],
        out_specs=pl.BlockSpec((1, _LT, s), lambda bi, li: (bi, li, 0)),
        out_shape=jax.ShapeDtypeStruct((b, l, s), jnp.float32),
    )(xt3, output, qt0, qt0t, ratet)


# speedup vs baseline: 1.0841x; 1.0841x over previous
import jax
import jax.numpy as jnp
from jax import lax
from jax.experimental import pallas as pl

_EPS = 1e-06
_LT = 1024


def _split_bf16(x):
    hi = x.astype(jnp.bfloat16)
    lo = (x - hi.astype(jnp.float32)).astype(jnp.bfloat16)
    return hi, lo


def _body(xt_ref, out_ref, qt0_ref, qt0t_ref, ratet_ref, o_ref):
    lt, s = out_ref.shape[1], out_ref.shape[2]
    f32 = jnp.float32
    xt = xt_ref[0, 0, :]
    iota = lax.broadcasted_iota(jnp.int32, (lt, s), 1)
    oh16 = (xt[:, None] == iota).astype(jnp.bfloat16)

    recip = (1.0 / (qt0t_ref[0] + _EPS)).astype(jnp.bfloat16)
    diag = lax.broadcasted_iota(jnp.int32, (s, s), 0) == \
        lax.broadcasted_iota(jnp.int32, (s, s), 1)
    ratez = jnp.where(diag, 0.0, ratet_ref[0]).astype(jnp.bfloat16)

    recip_g = jnp.dot(oh16, recip, preferred_element_type=f32)
    fwd_g = jnp.dot(oh16, ratez, preferred_element_type=f32)
    d16 = (out_ref[0] * recip_g).astype(jnp.bfloat16)
    score = jnp.dot(d16, qt0_ref[0].astype(jnp.bfloat16),
                    preferred_element_type=f32)
    o_ref[0] = fwd_g * score


def kernel(output, xt, t, qt0, rate):
    del t
    b, l, s = output.shape
    nb = l // _LT
    xt3 = xt.reshape(b * nb, 1, _LT)
    qt0t = qt0.swapaxes(1, 2)
    ratet = rate.swapaxes(1, 2)
    return pl.pallas_call(
        _body,
        grid=(b, nb),
        in_specs=[
            pl.BlockSpec((1, 1, _LT), lambda bi, li: (bi * nb + li, 0, 0)),
            pl.BlockSpec((1, _LT, s), lambda bi, li: (bi, li, 0)),
            pl.BlockSpec((1, s, s), lambda bi, li: (bi, 0, 0)),
            pl.BlockSpec((1, s, s), lambda bi, li: (bi, 0, 0)),
            pl.BlockSpec((1, s, s), lambda bi, li: (bi, 0, 0)),
        ],
        out_specs=pl.BlockSpec((1, _LT, s), lambda bi, li: (bi, li, 0)),
        out_shape=jax.ShapeDtypeStruct((b, l, s), jnp.float32),
    )(xt3, output, qt0, qt0t, ratet)


# speedup vs baseline: 1.4797x; 1.3649x over previous
import jax
import jax.numpy as jnp
from jax import lax
from jax.experimental import pallas as pl

_EPS = 1e-06
_LT = 3072


def _split_bf16(x):
    hi = x.astype(jnp.bfloat16)
    lo = (x - hi.astype(jnp.float32)).astype(jnp.bfloat16)
    return hi, lo


def _body(xt_ref, out_ref, qt0_ref, qt0t_ref, ratet_ref, o_ref):
    lt, s = out_ref.shape[1], out_ref.shape[2]
    f32 = jnp.float32
    xt = xt_ref[0, 0, :]
    iota = lax.broadcasted_iota(jnp.int32, (lt, s), 1)
    oh16 = (xt[:, None] == iota).astype(jnp.bfloat16)

    recip = (1.0 / (qt0t_ref[0] + _EPS)).astype(jnp.bfloat16)
    diag = lax.broadcasted_iota(jnp.int32, (s, s), 0) == \
        lax.broadcasted_iota(jnp.int32, (s, s), 1)
    ratez = jnp.where(diag, 0.0, ratet_ref[0]).astype(jnp.bfloat16)

    recip_g = jnp.dot(oh16, recip, preferred_element_type=f32)
    fwd_g = jnp.dot(oh16, ratez, preferred_element_type=f32)
    d16 = (out_ref[0] * recip_g).astype(jnp.bfloat16)
    score = jnp.dot(d16, qt0_ref[0].astype(jnp.bfloat16),
                    preferred_element_type=f32)
    o_ref[0] = fwd_g * score


def kernel(output, xt, t, qt0, rate):
    del t
    b, l, s = output.shape
    nb = l // _LT
    xt3 = xt.reshape(b * nb, 1, _LT)
    qt0t = qt0.swapaxes(1, 2)
    ratet = rate.swapaxes(1, 2)
    return pl.pallas_call(
        _body,
        grid=(b, nb),
        in_specs=[
            pl.BlockSpec((1, 1, _LT), lambda bi, li: (bi * nb + li, 0, 0)),
            pl.BlockSpec((1, _LT, s), lambda bi, li: (bi, li, 0)),
            pl.BlockSpec((1, s, s), lambda bi, li: (bi, 0, 0)),
            pl.BlockSpec((1, s, s), lambda bi, li: (bi, 0, 0)),
            pl.BlockSpec((1, s, s), lambda bi, li: (bi, 0, 0)),
        ],
        out_specs=pl.BlockSpec((1, _LT, s), lambda bi, li: (bi, li, 0)),
        out_shape=jax.ShapeDtypeStruct((b, l, s), jnp.float32),
    )(xt3, output, qt0, qt0t, ratet)


# speedup vs baseline: 1.4854x; 1.0038x over previous
import jax
import jax.numpy as jnp
from jax import lax
from jax.experimental import pallas as pl
from jax.experimental.pallas import tpu as pltpu

_EPS = 1e-06
_LT = 3072


def _split_bf16(x):
    hi = x.astype(jnp.bfloat16)
    lo = (x - hi.astype(jnp.float32)).astype(jnp.bfloat16)
    return hi, lo


def _body(xt_ref, out_ref, qt0_ref, qt0t_ref, ratet_ref, o_ref):
    lt, s = out_ref.shape[1], out_ref.shape[2]
    f32 = jnp.float32
    xt = xt_ref[0, 0, :]
    iota = lax.broadcasted_iota(jnp.int32, (lt, s), 1)
    oh16 = (xt[:, None] == iota).astype(jnp.bfloat16)

    recip = (1.0 / (qt0t_ref[0] + _EPS)).astype(jnp.bfloat16)
    diag = lax.broadcasted_iota(jnp.int32, (s, s), 0) == \
        lax.broadcasted_iota(jnp.int32, (s, s), 1)
    ratez = jnp.where(diag, 0.0, ratet_ref[0]).astype(jnp.bfloat16)

    recip_g = jnp.dot(oh16, recip, preferred_element_type=f32)
    fwd_g = jnp.dot(oh16, ratez, preferred_element_type=f32)
    d16 = (out_ref[0] * recip_g).astype(jnp.bfloat16)
    score = jnp.dot(d16, qt0_ref[0].astype(jnp.bfloat16),
                    preferred_element_type=f32)
    o_ref[0] = fwd_g * score


def kernel(output, xt, t, qt0, rate):
    del t
    b, l, s = output.shape
    nb = l // _LT
    xt3 = xt.reshape(b * nb, 1, _LT)
    qt0t = qt0.swapaxes(1, 2)
    ratet = rate.swapaxes(1, 2)
    return pl.pallas_call(
        _body,
        grid=(b, nb),
        in_specs=[
            pl.BlockSpec((1, 1, _LT), lambda bi, li: (bi * nb + li, 0, 0)),
            pl.BlockSpec((1, _LT, s), lambda bi, li: (bi, li, 0)),
            pl.BlockSpec((1, s, s), lambda bi, li: (bi, 0, 0)),
            pl.BlockSpec((1, s, s), lambda bi, li: (bi, 0, 0)),
            pl.BlockSpec((1, s, s), lambda bi, li: (bi, 0, 0)),
        ],
        out_specs=pl.BlockSpec((1, _LT, s), lambda bi, li: (bi, li, 0)),
        out_shape=jax.ShapeDtypeStruct((b, l, s), jnp.float32),
        compiler_params=pltpu.CompilerParams(
            dimension_semantics=("parallel", "arbitrary")),
    )(xt3, output, qt0, qt0t, ratet)


# speedup vs baseline: 1.8212x; 1.2261x over previous
import jax
import jax.numpy as jnp
from jax import lax
from jax.experimental import pallas as pl
from jax.experimental.pallas import tpu as pltpu

_EPS = 1e-06
_LT = 3072


def _body(xt_ref, out_ref, qt0_ref, rate_ref, o_ref):
    lt, s = out_ref.shape[1], out_ref.shape[2]
    f32, bf16 = jnp.float32, jnp.bfloat16
    xt = xt_ref[0, 0, :]
    iota = lax.broadcasted_iota(jnp.int32, (lt, s), 1)
    oh16 = (xt[:, None] == iota).astype(bf16)

    recip = (1.0 / (qt0_ref[0] + _EPS)).astype(bf16)
    diag = lax.broadcasted_iota(jnp.int32, (s, s), 0) == \
        lax.broadcasted_iota(jnp.int32, (s, s), 1)
    ratez = jnp.where(diag, 0.0, rate_ref[0]).astype(bf16)

    dn_t = (((1,), (1,)), ((), ()))
    recip_g = lax.dot_general(oh16, recip, dn_t, preferred_element_type=f32)
    fwd_g = lax.dot_general(oh16, ratez, dn_t, preferred_element_type=f32)
    d16 = (out_ref[0] * recip_g).astype(bf16)
    score = jnp.dot(d16, qt0_ref[0].astype(bf16), preferred_element_type=f32)
    o_ref[0] = fwd_g * score


def kernel(output, xt, t, qt0, rate):
    del t
    b, l, s = output.shape
    nb = l // _LT
    xt3 = xt.reshape(b * nb, 1, _LT)
    return pl.pallas_call(
        _body,
        grid=(b, nb),
        in_specs=[
            pl.BlockSpec((1, 1, _LT), lambda bi, li: (bi * nb + li, 0, 0)),
            pl.BlockSpec((1, _LT, s), lambda bi, li: (bi, li, 0)),
            pl.BlockSpec((1, s, s), lambda bi, li: (bi, 0, 0)),
            pl.BlockSpec((1, s, s), lambda bi, li: (bi, 0, 0)),
        ],
        out_specs=pl.BlockSpec((1, _LT, s), lambda bi, li: (bi, li, 0)),
        out_shape=jax.ShapeDtypeStruct((b, l, s), jnp.float32),
        compiler_params=pltpu.CompilerParams(
            dimension_semantics=("parallel", "arbitrary")),
    )(xt3, output, qt0, rate)
